# baseline (speedup 1.0000x reference)
import functools

import jax
import jax.numpy as jnp
from jax import lax
from jax.experimental import pallas as pl
from jax.experimental.pallas import tpu as pltpu
from jax.experimental.pallas import tpu_sc as plsc

_NC = 2
_NS = 16
_NW = _NC * _NS
_CH = 112
_L = 16


def _sc_sum_pool(xp, emb_table):
    b_total, nch, ch = xp.shape
    _, d = emb_table.shape
    bpw = b_total // _NW
    nd = d // _L
    mesh = plsc.VectorSubcoreMesh(core_axis_name="c", subcore_axis_name="s")

    nbuf = 2 * nch

    @functools.partial(
        pl.kernel,
        out_type=jax.ShapeDtypeStruct((b_total, d), jnp.float32),
        mesh=mesh,
        compiler_params=pltpu.CompilerParams(use_tc_tiling_on_sc=False),
        scratch_types=(
            [pltpu.VMEM((bpw, nch, ch), jnp.int32)]
            + [pltpu.VMEM((ch, d), jnp.float32) for _ in range(nbuf)]
            + [pltpu.VMEM((bpw, d), jnp.float32)]
            + [pltpu.SemaphoreType.DMA for _ in range(nbuf)]
        ),
    )
    def pool(xp_hbm, emb_hbm, out_hbm, *refs):
        idx_v = refs[0]
        bufs = refs[1:1 + nbuf]
        out_v = refs[1 + nbuf]
        sems = refs[2 + nbuf:2 + 2 * nbuf]

        wid = lax.axis_index("s") * _NC + lax.axis_index("c")
        base = wid * bpw
        pltpu.sync_copy(xp_hbm.at[pl.ds(base, bpw)], idx_v)

        def reduce_buf(buf, acc):
            def rb(s, a):
                return tuple(a[j] + buf[s, pl.ds(j * _L, _L)] for j in range(nd))
            return lax.fori_loop(0, ch, rb, acc)

        def finish_row(r, row_bufs, handles):
            acc = tuple(jnp.zeros((_L,), jnp.float32) for _ in range(nd))
            for c in range(nch):
                handles[c].wait()
                acc = reduce_buf(row_bufs[c], acc)
            for j in range(nd):
                out_v[r, pl.ds(j * _L, _L)] = acc[j]

        def pair_body(i, carry):
            ra = 2 * i
            rb = 2 * i + 1
            ha = [pltpu.async_copy(emb_hbm.at[idx_v.at[ra, c]], bufs[c], sems[c])
                  for c in range(nch)]
            hb = [pltpu.async_copy(emb_hbm.at[idx_v.at[rb, c]],
                                   bufs[nch + c], sems[nch + c])
                  for c in range(nch)]
            finish_row(ra, bufs[:nch], ha)
            finish_row(rb, bufs[nch:], hb)
            return carry

        lax.fori_loop(0, bpw // 2, pair_body, jnp.int32(0))
        pltpu.sync_copy(out_v, out_hbm.at[pl.ds(base, bpw)])

    return pool(xp, emb_table)


def _tc_mlp(sums, x, row0, W1, b1, W2p, b2p, seq_pad):
    b_total, _ = sums.shape
    h = W1.shape[1]
    o = W2p.shape[1]

    def body(s_ref, x_ref, r0_ref, w1_ref, b1_ref, w2_ref, b2_ref, o_ref):
        lenf = jnp.sum((x_ref[...] != 0).astype(jnp.float32), axis=1,
                       keepdims=True)
        pad_cnt = seq_pad - lenf
        avg = (s_ref[...] - pad_cnt * r0_ref[...]) / jnp.maximum(lenf, 1.0)
        hh = jnp.dot(avg, w1_ref[...], preferred_element_type=jnp.float32)
        hh = jnp.maximum(hh + b1_ref[...], 0.0)
        o_ref[...] = jnp.dot(hh, w2_ref[...],
                             preferred_element_type=jnp.float32) + b2_ref[...]

    return pl.pallas_call(
        body,
        out_shape=jax.ShapeDtypeStruct((b_total, o), jnp.float32),
    )(sums, x, row0, W1, b1.reshape(1, h), W2p, b2p.reshape(1, o))


def kernel(x, emb_table, W1, b1, W2, b2):
    x = x.astype(jnp.int32)
    b_total, s = x.shape
    nch = -(-s // _CH)
    pad = nch * _CH - s
    xp = jnp.pad(x, ((0, 0), (0, pad))).reshape(b_total, nch, _CH)
    sums = _sc_sum_pool(xp, emb_table)
    o = 128
    w2p = jnp.pad(W2, ((0, 0), (0, o - W2.shape[1])))
    b2p = jnp.pad(b2, (0, o - b2.shape[0]))
    row0 = emb_table[0:1]
    out = _tc_mlp(sums, x, row0, W1, b1, w2p, b2p, float(nch * _CH))
    return out[:, : W2.shape[1]]
